# baseline (speedup 1.0000x reference)
import functools

import jax
import jax.numpy as jnp
from jax import lax
from jax.experimental import pallas as pl
from jax.experimental.pallas import tpu as pltpu
from jax.experimental.pallas import tpu_sc as plsc

Q, K, C, TOPK = 4096, 16384, 256, 16


BQ = 256
CHK = 512
NCH = K // CHK

_INF = float("inf")


def _topk_body(q_ref, pt_ref, qq_ref, pp_ref, idx_ref, d2_scr):
    def compute(c, carry):
        cross = jnp.dot(q_ref[...], pt_ref[c])
        d2_scr[c] = qq_ref[...] - 2.0 * cross + pp_ref[c]
        return carry

    lax.fori_loop(0, NCH, compute, 0, unroll=False)

    cols0 = lax.broadcasted_iota(jnp.int32, (BQ, CHK), 1)
    out_lane = lax.broadcasted_iota(jnp.int32, (BQ, TOPK), 1)

    def extract(j, carry):
        m_prev, a_prev, acc = carry

        def chunk(c, mc):
            m, am = mc
            blk = d2_scr[c]
            cols = cols0 + c * CHK
            ok = (blk > m_prev) | ((blk == m_prev) & (cols > a_prev))
            cand = jnp.where(ok, blk, _INF)
            cm = jnp.min(cand, axis=1, keepdims=True)
            ca = jnp.min(jnp.where(cand == cm, cols, K), axis=1,
                         keepdims=True)
            better = (cm < m) | ((cm == m) & (ca < am))
            return (jnp.where(better, cm, m), jnp.where(better, ca, am))

        m, am = lax.fori_loop(0, NCH, chunk,
                              (jnp.full((BQ, 1), _INF, jnp.float32),
                               jnp.full((BQ, 1), K, jnp.int32)),
                              unroll=False)
        acc = jnp.where(out_lane == j, jnp.broadcast_to(am, (BQ, TOPK)), acc)
        return (m, am, acc)

    _, _, acc = lax.fori_loop(
        0, TOPK, extract,
        (jnp.full((BQ, 1), -_INF, jnp.float32),
         jnp.full((BQ, 1), -1, jnp.int32),
         jnp.zeros((BQ, TOPK), jnp.int32)),
        unroll=False)
    idx_ref[...] = acc


def _topk_call(queries, ptc, qq, ppc):
    return pl.pallas_call(
        _topk_body,
        grid=(Q // BQ,),
        in_specs=[
            pl.BlockSpec((BQ, 3), lambda i: (i, 0)),
            pl.BlockSpec((NCH, 3, CHK), lambda i: (0, 0, 0)),
            pl.BlockSpec((BQ, 1), lambda i: (i, 0)),
            pl.BlockSpec((NCH, 1, CHK), lambda i: (0, 0, 0)),
        ],
        out_specs=pl.BlockSpec((BQ, TOPK), lambda i: (i, 0)),
        out_shape=jax.ShapeDtypeStruct((Q, TOPK), jnp.int32),
        scratch_shapes=[pltpu.VMEM((NCH, BQ, CHK), jnp.float32)],
    )(queries, ptc, qq, ppc)


_NW = 32
_ROWS_PER_W = (TOPK * Q) // _NW
_GCH = 128


def _gather_body(table_hbm, idx_hbm, out_hbm, idx_v, rows_v, sem):
    wid = lax.axis_index("s") * 2 + lax.axis_index("c")
    base0 = wid * _ROWS_PER_W

    def step(t, carry):
        base = base0 + t * _GCH
        pltpu.sync_copy(idx_hbm.at[pl.ds(base, _GCH)], idx_v)
        pltpu.async_copy(table_hbm.at[idx_v], rows_v, sem).wait()
        pltpu.sync_copy(rows_v, out_hbm.at[pl.ds(base, _GCH)])
        return carry

    lax.fori_loop(0, _ROWS_PER_W // _GCH, step, 0)


_gather = pl.kernel(
    _gather_body,
    out_type=jax.ShapeDtypeStruct((TOPK * Q, C), jnp.float32),
    mesh=plsc.VectorSubcoreMesh(core_axis_name="c", subcore_axis_name="s"),
    scratch_types=[
        pltpu.VMEM((_GCH,), jnp.int32),
        pltpu.VMEM((_GCH, C), jnp.float32),
        pltpu.SemaphoreType.DMA,
    ],
)


BQ3 = 256
NB3 = Q // BQ3
N3 = BQ3 * C


def _mlp_body(g_ref, w1t_ref, b1_ref, w2t_ref, b2_ref, o_ref):
    h = jnp.dot(w1t_ref[...], g_ref[...])
    h = jnp.maximum(h + b1_ref[...], 0.0)
    o = jnp.dot(w2t_ref[...], h) + b2_ref[...]
    o_ref[...] = o.reshape(1, 1, N3)


def _mlp_call(gflat, w1t, b1c, w2t, b2c):
    return pl.pallas_call(
        _mlp_body,
        grid=(NB3,),
        in_specs=[
            pl.BlockSpec((TOPK, N3), lambda i: (0, i)),
            pl.BlockSpec((TOPK, TOPK), lambda i: (0, 0)),
            pl.BlockSpec((TOPK, 1), lambda i: (0, 0)),
            pl.BlockSpec((1, TOPK), lambda i: (0, 0)),
            pl.BlockSpec((1, 1), lambda i: (0, 0)),
        ],
        out_specs=pl.BlockSpec((1, 1, N3), lambda i: (i, 0, 0)),
        out_shape=jax.ShapeDtypeStruct((NB3, 1, N3), jnp.float32),
    )(gflat, w1t, b1c, w2t, b2c)


def kernel(queries, keys, point_feats, bev_feats, W1, b1, W2, b2, R, t):
    proj = keys @ R.T + t
    qq = jnp.sum(queries * queries, axis=1, keepdims=True)
    pp = jnp.sum(proj * proj, axis=1)[None, :]
    ptc = proj.T.reshape(3, NCH, CHK).transpose(1, 0, 2)
    ppc = pp.reshape(1, NCH, CHK).transpose(1, 0, 2)

    idx = _topk_call(queries, ptc, qq, ppc)
    idx_flat = idx.T.reshape(TOPK * Q)

    g = _gather(point_feats, idx_flat)
    gflat = g.reshape(TOPK, Q * C)

    hb = _mlp_call(gflat, W1.T, b1.reshape(TOPK, 1),
                   W2.T, b2.reshape(1, 1))
    h = hb.reshape(Q, C)
    return jnp.concatenate([bev_feats, h], axis=-1)


# speedup vs baseline: 2.3547x; 2.3547x over previous
import functools

import jax
import jax.numpy as jnp
from jax import lax
from jax.experimental import pallas as pl
from jax.experimental.pallas import tpu as pltpu
from jax.experimental.pallas import tpu_sc as plsc

Q, K, C, TOPK = 4096, 16384, 256, 16


BQ = 256
CHK = 512
NCH = K // CHK

_INF = float("inf")


NSL = CHK // 128
NSEG = 128
RMAX = TOPK


def _topk_body(q_ref, pt_ref, qq_ref, pp_ref, idx_ref, d2_scr, cv_scr, cc_scr):
    def compute(c, carry):
        cross = jnp.dot(q_ref[...], pt_ref[c])
        d2_scr[c] = qq_ref[...] - 2.0 * cross + pp_ref[c]
        return carry

    lax.fori_loop(0, NCH, compute, 0, unroll=False)

    lane = lax.broadcasted_iota(jnp.int32, (BQ, NSEG), 1)
    out_lane = lax.broadcasted_iota(jnp.int32, (BQ, TOPK), 1)
    inf_v = jnp.full((BQ, NSEG), _INF, jnp.float32)
    big_c = jnp.full((BQ, NSEG), K, jnp.int32)
    for r in range(RMAX):
        cv_scr[r] = inf_v
        cc_scr[r] = big_c

    def round_cond(carry):
        r, _, _, done = carry
        return jnp.logical_and(r < RMAX, jnp.logical_not(done))

    def round_body(carry):
        r, f_v, f_c, _ = carry

        def chunk(c, mc):
            m, am = mc
            d2c = d2_scr[c]
            for s in range(NSL):
                blk = d2c[:, s * 128:(s + 1) * 128]
                cols = lane + (c * CHK + s * 128)
                ok = (blk > f_v) | ((blk == f_v) & (cols > f_c))
                cand = jnp.where(ok, blk, _INF)
                lt = (cand < m) | ((cand == m) & (cols < am))
                m = jnp.where(lt, cand, m)
                am = jnp.where(lt, cols, am)
            return (m, am)

        m, am = lax.fori_loop(0, NCH, chunk, (inf_v, big_c), unroll=False)
        cv_scr[r] = m
        cc_scr[r] = am
        fv = jnp.min(m, axis=1, keepdims=True)
        fc = jnp.min(jnp.where(m == fv, am, K), axis=1, keepdims=True)

        def count(rr, cnt):
            v = cv_scr[rr]
            cc = cc_scr[rr]
            hit = (v < fv) | ((v == fv) & (cc <= fc))
            return cnt + jnp.sum(hit.astype(jnp.int32), axis=1, keepdims=True)

        cnt = lax.fori_loop(0, r + 1, count,
                            jnp.zeros((BQ, 1), jnp.int32), unroll=False)
        done = jnp.min(cnt) >= TOPK
        return (r + 1, m, am, done)

    r_fin, _, _, _ = lax.while_loop(
        round_cond, round_body,
        (jnp.int32(0),
         jnp.full((BQ, NSEG), -_INF, jnp.float32),
         jnp.full((BQ, NSEG), -1, jnp.int32),
         jnp.bool_(False)))

    def extract(j, carry):
        m_prev, a_prev, acc = carry

        def cslice(rr, mc):
            m, am = mc
            v = cv_scr[rr]
            cc = cc_scr[rr]
            ok = (v > m_prev) | ((v == m_prev) & (cc > a_prev))
            cand = jnp.where(ok, v, _INF)
            lt = (cand < m) | ((cand == m) & (cc < am))
            return (jnp.where(lt, cand, m), jnp.where(lt, cc, am))

        mv, mc_ = lax.fori_loop(0, r_fin, cslice, (inf_v, big_c),
                                unroll=False)
        m = jnp.min(mv, axis=1, keepdims=True)
        am = jnp.min(jnp.where(mv == m, mc_, K), axis=1, keepdims=True)
        acc = jnp.where(out_lane == j, jnp.broadcast_to(am, (BQ, TOPK)), acc)
        return (m, am, acc)

    _, _, acc = lax.fori_loop(
        0, TOPK, extract,
        (jnp.full((BQ, 1), -_INF, jnp.float32),
         jnp.full((BQ, 1), -1, jnp.int32),
         jnp.zeros((BQ, TOPK), jnp.int32)),
        unroll=False)
    idx_ref[...] = acc


def _topk_call(queries, ptc, qq, ppc):
    return pl.pallas_call(
        _topk_body,
        grid=(Q // BQ,),
        in_specs=[
            pl.BlockSpec((BQ, 3), lambda i: (i, 0)),
            pl.BlockSpec((NCH, 3, CHK), lambda i: (0, 0, 0)),
            pl.BlockSpec((BQ, 1), lambda i: (i, 0)),
            pl.BlockSpec((NCH, 1, CHK), lambda i: (0, 0, 0)),
        ],
        out_specs=pl.BlockSpec((BQ, TOPK), lambda i: (i, 0)),
        out_shape=jax.ShapeDtypeStruct((Q, TOPK), jnp.int32),
        scratch_shapes=[pltpu.VMEM((NCH, BQ, CHK), jnp.float32),
                        pltpu.VMEM((RMAX, BQ, NSEG), jnp.float32),
                        pltpu.VMEM((RMAX, BQ, NSEG), jnp.int32)],
    )(queries, ptc, qq, ppc)


_NW = 32
_ROWS_PER_W = (TOPK * Q) // _NW
_GCH = 128


def _gather_body(table_hbm, idx_hbm, out_hbm, idx_v, rows_v, sem):
    wid = lax.axis_index("s") * 2 + lax.axis_index("c")
    base0 = wid * _ROWS_PER_W

    def step(t, carry):
        base = base0 + t * _GCH
        pltpu.sync_copy(idx_hbm.at[pl.ds(base, _GCH)], idx_v)
        pltpu.async_copy(table_hbm.at[idx_v], rows_v, sem).wait()
        pltpu.sync_copy(rows_v, out_hbm.at[pl.ds(base, _GCH)])
        return carry

    lax.fori_loop(0, _ROWS_PER_W // _GCH, step, 0)


_gather = pl.kernel(
    _gather_body,
    out_type=jax.ShapeDtypeStruct((TOPK * Q, C), jnp.float32),
    mesh=plsc.VectorSubcoreMesh(core_axis_name="c", subcore_axis_name="s"),
    scratch_types=[
        pltpu.VMEM((_GCH,), jnp.int32),
        pltpu.VMEM((_GCH, C), jnp.float32),
        pltpu.SemaphoreType.DMA,
    ],
)


BQ3 = 256
NB3 = Q // BQ3
N3 = BQ3 * C


def _mlp_body(g_ref, w1t_ref, b1_ref, w2t_ref, b2_ref, o_ref):
    h = jnp.dot(w1t_ref[...], g_ref[...])
    h = jnp.maximum(h + b1_ref[...], 0.0)
    o = jnp.dot(w2t_ref[...], h) + b2_ref[...]
    o_ref[...] = o.reshape(1, 1, N3)


def _mlp_call(gflat, w1t, b1c, w2t, b2c):
    return pl.pallas_call(
        _mlp_body,
        grid=(NB3,),
        in_specs=[
            pl.BlockSpec((TOPK, N3), lambda i: (0, i)),
            pl.BlockSpec((TOPK, TOPK), lambda i: (0, 0)),
            pl.BlockSpec((TOPK, 1), lambda i: (0, 0)),
            pl.BlockSpec((1, TOPK), lambda i: (0, 0)),
            pl.BlockSpec((1, 1), lambda i: (0, 0)),
        ],
        out_specs=pl.BlockSpec((1, 1, N3), lambda i: (i, 0, 0)),
        out_shape=jax.ShapeDtypeStruct((NB3, 1, N3), jnp.float32),
    )(gflat, w1t, b1c, w2t, b2c)


def kernel(queries, keys, point_feats, bev_feats, W1, b1, W2, b2, R, t):
    proj = keys @ R.T + t
    qq = jnp.sum(queries * queries, axis=1, keepdims=True)
    pp = jnp.sum(proj * proj, axis=1)[None, :]
    ptc = proj.T.reshape(3, NCH, CHK).transpose(1, 0, 2)
    ppc = pp.reshape(1, NCH, CHK).transpose(1, 0, 2)

    idx = _topk_call(queries, ptc, qq, ppc)
    idx_flat = idx.T.reshape(TOPK * Q)

    g = _gather(point_feats, idx_flat)
    gflat = g.reshape(TOPK, Q * C)

    hb = _mlp_call(gflat, W1.T, b1.reshape(TOPK, 1),
                   W2.T, b2.reshape(1, 1))
    h = hb.reshape(Q, C)
    return jnp.concatenate([bev_feats, h], axis=-1)


# speedup vs baseline: 2.3890x; 1.0146x over previous
import functools

import jax
import jax.numpy as jnp
from jax import lax
from jax.experimental import pallas as pl
from jax.experimental.pallas import tpu as pltpu
from jax.experimental.pallas import tpu_sc as plsc

Q, K, C, TOPK = 4096, 16384, 256, 16


BQ = 256
CHK = 512
NCH = K // CHK

_INF = float("inf")


NSL = CHK // 128
NSEG = 128
RMAX = TOPK


def _topk_body(q_ref, pt_ref, qq_ref, pp_ref, idx_ref, d2_scr, cv_scr, cc_scr):
    def compute(c, carry):
        cross = jnp.dot(q_ref[...], pt_ref[c])
        d2_scr[c] = qq_ref[...] - 2.0 * cross + pp_ref[c]
        return carry

    lax.fori_loop(0, NCH, compute, 0, unroll=False)

    lane = lax.broadcasted_iota(jnp.int32, (BQ, NSEG), 1).astype(jnp.float32)
    out_lane = lax.broadcasted_iota(jnp.int32, (BQ, TOPK), 1)
    inf_v = jnp.full((BQ, NSEG), _INF, jnp.float32)
    big_c = jnp.full((BQ, NSEG), float(K), jnp.float32)
    for r in range(RMAX):
        cv_scr[r] = inf_v
        cc_scr[r] = big_c

    def round_cond(carry):
        r, _, _, done = carry
        return jnp.logical_and(r < RMAX, jnp.logical_not(done))

    def round_body(carry):
        r, f_v, f_c, _ = carry

        def chunk(c, mc):
            m, am = mc
            d2c = d2_scr[c]
            for s in range(NSL):
                blk = d2c[:, s * 128:(s + 1) * 128]
                cols = lane + (c * CHK + s * 128).astype(jnp.float32)
                ok = (blk > f_v) | ((blk == f_v) & (cols > f_c))
                cand = jnp.where(ok, blk, _INF)
                lt = (cand < m) | ((cand == m) & (cols < am))
                m = jnp.where(lt, cand, m)
                am = jnp.where(lt, cols, am)
            return (m, am)

        m, am = lax.fori_loop(0, NCH, chunk, (inf_v, big_c), unroll=False)
        cv_scr[r] = m
        cc_scr[r] = am
        fv = jnp.min(m, axis=1, keepdims=True)
        fc = jnp.min(jnp.where(m == fv, am, float(K)), axis=1, keepdims=True)

        def count(rr, cnt):
            v = cv_scr[rr]
            cc = cc_scr[rr]
            hit = (v < fv) | ((v == fv) & (cc <= fc))
            return cnt + jnp.sum(hit.astype(jnp.int32), axis=1, keepdims=True)

        cnt = lax.fori_loop(0, r + 1, count,
                            jnp.zeros((BQ, 1), jnp.int32), unroll=False)
        done = jnp.min(cnt) >= TOPK
        return (r + 1, m, am, done)

    r_fin, _, _, _ = lax.while_loop(
        round_cond, round_body,
        (jnp.int32(0),
         jnp.full((BQ, NSEG), -_INF, jnp.float32),
         jnp.full((BQ, NSEG), -1.0, jnp.float32),
         jnp.bool_(False)))

    def extract(j, carry):
        m_prev, a_prev, acc = carry

        def cslice(rr, mc):
            m, am = mc
            v = cv_scr[rr]
            cc = cc_scr[rr]
            ok = (v > m_prev) | ((v == m_prev) & (cc > a_prev))
            cand = jnp.where(ok, v, _INF)
            lt = (cand < m) | ((cand == m) & (cc < am))
            return (jnp.where(lt, cand, m), jnp.where(lt, cc, am))

        mv, mc_ = lax.fori_loop(0, r_fin, cslice, (inf_v, big_c),
                                unroll=False)
        m = jnp.min(mv, axis=1, keepdims=True)
        am = jnp.min(jnp.where(mv == m, mc_, float(K)), axis=1, keepdims=True)
        acc = jnp.where(out_lane == j, jnp.broadcast_to(am, (BQ, TOPK)), acc)
        return (m, am, acc)

    _, _, acc = lax.fori_loop(
        0, TOPK, extract,
        (jnp.full((BQ, 1), -_INF, jnp.float32),
         jnp.full((BQ, 1), -1.0, jnp.float32),
         jnp.zeros((BQ, TOPK), jnp.float32)),
        unroll=False)
    idx_ref[...] = acc.astype(jnp.int32)


def _topk_call(queries, ptc, qq, ppc):
    return pl.pallas_call(
        _topk_body,
        grid=(Q // BQ,),
        in_specs=[
            pl.BlockSpec((BQ, 3), lambda i: (i, 0)),
            pl.BlockSpec((NCH, 3, CHK), lambda i: (0, 0, 0)),
            pl.BlockSpec((BQ, 1), lambda i: (i, 0)),
            pl.BlockSpec((NCH, 1, CHK), lambda i: (0, 0, 0)),
        ],
        out_specs=pl.BlockSpec((BQ, TOPK), lambda i: (i, 0)),
        out_shape=jax.ShapeDtypeStruct((Q, TOPK), jnp.int32),
        scratch_shapes=[pltpu.VMEM((NCH, BQ, CHK), jnp.float32),
                        pltpu.VMEM((RMAX, BQ, NSEG), jnp.float32),
                        pltpu.VMEM((RMAX, BQ, NSEG), jnp.float32)],
    )(queries, ptc, qq, ppc)


_NW = 32
_ROWS_PER_W = (TOPK * Q) // _NW
_GCH = 128


def _gather_body(table_hbm, idx_hbm, out_hbm, idx_v, rows_v, sem):
    wid = lax.axis_index("s") * 2 + lax.axis_index("c")
    base0 = wid * _ROWS_PER_W

    def step(t, carry):
        base = base0 + t * _GCH
        pltpu.sync_copy(idx_hbm.at[pl.ds(base, _GCH)], idx_v)
        pltpu.async_copy(table_hbm.at[idx_v], rows_v, sem).wait()
        pltpu.sync_copy(rows_v, out_hbm.at[pl.ds(base, _GCH)])
        return carry

    lax.fori_loop(0, _ROWS_PER_W // _GCH, step, 0)


_gather = pl.kernel(
    _gather_body,
    out_type=jax.ShapeDtypeStruct((TOPK * Q, C), jnp.float32),
    mesh=plsc.VectorSubcoreMesh(core_axis_name="c", subcore_axis_name="s"),
    scratch_types=[
        pltpu.VMEM((_GCH,), jnp.int32),
        pltpu.VMEM((_GCH, C), jnp.float32),
        pltpu.SemaphoreType.DMA,
    ],
)


BQ3 = 256
NB3 = Q // BQ3
N3 = BQ3 * C


def _mlp_body(g_ref, w1t_ref, b1_ref, w2t_ref, b2_ref, o_ref):
    h = jnp.dot(w1t_ref[...], g_ref[...])
    h = jnp.maximum(h + b1_ref[...], 0.0)
    o = jnp.dot(w2t_ref[...], h) + b2_ref[...]
    o_ref[...] = o.reshape(1, 1, N3)


def _mlp_call(gflat, w1t, b1c, w2t, b2c):
    return pl.pallas_call(
        _mlp_body,
        grid=(NB3,),
        in_specs=[
            pl.BlockSpec((TOPK, N3), lambda i: (0, i)),
            pl.BlockSpec((TOPK, TOPK), lambda i: (0, 0)),
            pl.BlockSpec((TOPK, 1), lambda i: (0, 0)),
            pl.BlockSpec((1, TOPK), lambda i: (0, 0)),
            pl.BlockSpec((1, 1), lambda i: (0, 0)),
        ],
        out_specs=pl.BlockSpec((1, 1, N3), lambda i: (i, 0, 0)),
        out_shape=jax.ShapeDtypeStruct((NB3, 1, N3), jnp.float32),
    )(gflat, w1t, b1c, w2t, b2c)


def kernel(queries, keys, point_feats, bev_feats, W1, b1, W2, b2, R, t):
    proj = keys @ R.T + t
    qq = jnp.sum(queries * queries, axis=1, keepdims=True)
    pp = jnp.sum(proj * proj, axis=1)[None, :]
    ptc = proj.T.reshape(3, NCH, CHK).transpose(1, 0, 2)
    ppc = pp.reshape(1, NCH, CHK).transpose(1, 0, 2)

    idx = _topk_call(queries, ptc, qq, ppc)
    idx_flat = idx.T.reshape(TOPK * Q)

    g = _gather(point_feats, idx_flat)
    gflat = g.reshape(TOPK, Q * C)

    hb = _mlp_call(gflat, W1.T, b1.reshape(TOPK, 1),
                   W2.T, b2.reshape(1, 1))
    h = hb.reshape(Q, C)
    return jnp.concatenate([bev_feats, h], axis=-1)


# speedup vs baseline: 2.5458x; 1.0656x over previous
import functools

import jax
import jax.numpy as jnp
from jax import lax
from jax.experimental import pallas as pl
from jax.experimental.pallas import tpu as pltpu
from jax.experimental.pallas import tpu_sc as plsc

Q, K, C, TOPK = 4096, 16384, 256, 16


BQ = 128
CHK = 512
NCH = K // CHK

_INF = float("inf")


NSL = CHK // 128
NSEG = 128
RMAX = TOPK


def _topk_body(q_ref, pt_ref, qq_ref, pp_ref, idx_ref, d2_scr, cv_scr, cc_scr):
    def compute(c, carry):
        cross = jnp.dot(q_ref[...], pt_ref[c])
        d2_scr[c] = qq_ref[...] - 2.0 * cross + pp_ref[c]
        return carry

    lax.fori_loop(0, NCH, compute, 0, unroll=False)

    lane = lax.broadcasted_iota(jnp.int32, (BQ, NSEG), 1).astype(jnp.float32)
    out_lane = lax.broadcasted_iota(jnp.int32, (BQ, TOPK), 1)
    inf_v = jnp.full((BQ, NSEG), _INF, jnp.float32)
    big_c = jnp.full((BQ, NSEG), float(K), jnp.float32)
    for r in range(RMAX):
        cv_scr[r] = inf_v
        cc_scr[r] = big_c

    def round_cond(carry):
        r, _, _, done = carry
        return jnp.logical_and(r < RMAX, jnp.logical_not(done))

    def round_body(carry):
        r, f_v, f_c, _ = carry

        def chunk(c, mc):
            m, am = mc
            d2c = d2_scr[c]
            for s in range(NSL):
                blk = d2c[:, s * 128:(s + 1) * 128]
                cols = lane + (c * CHK + s * 128).astype(jnp.float32)
                ok = (blk > f_v) | ((blk == f_v) & (cols > f_c))
                cand = jnp.where(ok, blk, _INF)
                lt = (cand < m) | ((cand == m) & (cols < am))
                m = jnp.where(lt, cand, m)
                am = jnp.where(lt, cols, am)
            return (m, am)

        m, am = lax.fori_loop(0, NCH, chunk, (inf_v, big_c), unroll=False)
        cv_scr[r] = m
        cc_scr[r] = am
        fv = jnp.min(m, axis=1, keepdims=True)
        fc = jnp.min(jnp.where(m == fv, am, float(K)), axis=1, keepdims=True)

        def count(rr, cnt):
            v = cv_scr[rr]
            cc = cc_scr[rr]
            hit = (v < fv) | ((v == fv) & (cc <= fc))
            return cnt + jnp.sum(hit.astype(jnp.int32), axis=1, keepdims=True)

        cnt = lax.fori_loop(0, r + 1, count,
                            jnp.zeros((BQ, 1), jnp.int32), unroll=False)
        done = jnp.min(cnt) >= TOPK
        return (r + 1, m, am, done)

    r_fin, _, _, _ = lax.while_loop(
        round_cond, round_body,
        (jnp.int32(0),
         jnp.full((BQ, NSEG), -_INF, jnp.float32),
         jnp.full((BQ, NSEG), -1.0, jnp.float32),
         jnp.bool_(False)))

    def extract(j, carry):
        m_prev, a_prev, acc = carry

        def cslice(rr, mc):
            m, am = mc
            v = cv_scr[rr]
            cc = cc_scr[rr]
            ok = (v > m_prev) | ((v == m_prev) & (cc > a_prev))
            cand = jnp.where(ok, v, _INF)
            lt = (cand < m) | ((cand == m) & (cc < am))
            return (jnp.where(lt, cand, m), jnp.where(lt, cc, am))

        mv, mc_ = lax.fori_loop(0, r_fin, cslice, (inf_v, big_c),
                                unroll=False)
        m = jnp.min(mv, axis=1, keepdims=True)
        am = jnp.min(jnp.where(mv == m, mc_, float(K)), axis=1, keepdims=True)
        acc = jnp.where(out_lane == j, jnp.broadcast_to(am, (BQ, TOPK)), acc)
        return (m, am, acc)

    _, _, acc = lax.fori_loop(
        0, TOPK, extract,
        (jnp.full((BQ, 1), -_INF, jnp.float32),
         jnp.full((BQ, 1), -1.0, jnp.float32),
         jnp.zeros((BQ, TOPK), jnp.float32)),
        unroll=False)
    idx_ref[...] = acc.astype(jnp.int32)


def _topk_call(queries, ptc, qq, ppc):
    return pl.pallas_call(
        _topk_body,
        grid=(Q // BQ,),
        in_specs=[
            pl.BlockSpec((BQ, 3), lambda i: (i, 0)),
            pl.BlockSpec((NCH, 3, CHK), lambda i: (0, 0, 0)),
            pl.BlockSpec((BQ, 1), lambda i: (i, 0)),
            pl.BlockSpec((NCH, 1, CHK), lambda i: (0, 0, 0)),
        ],
        out_specs=pl.BlockSpec((BQ, TOPK), lambda i: (i, 0)),
        out_shape=jax.ShapeDtypeStruct((Q, TOPK), jnp.int32),
        scratch_shapes=[pltpu.VMEM((NCH, BQ, CHK), jnp.float32),
                        pltpu.VMEM((RMAX, BQ, NSEG), jnp.float32),
                        pltpu.VMEM((RMAX, BQ, NSEG), jnp.float32)],
    )(queries, ptc, qq, ppc)


_NW = 32
_ROWS_PER_W = (TOPK * Q) // _NW
_GCH = 128


def _gather_body(table_hbm, idx_hbm, out_hbm, idx_v, rows_v, sem):
    wid = lax.axis_index("s") * 2 + lax.axis_index("c")
    base0 = wid * _ROWS_PER_W

    def step(t, carry):
        base = base0 + t * _GCH
        pltpu.sync_copy(idx_hbm.at[pl.ds(base, _GCH)], idx_v)
        pltpu.async_copy(table_hbm.at[idx_v], rows_v, sem).wait()
        pltpu.sync_copy(rows_v, out_hbm.at[pl.ds(base, _GCH)])
        return carry

    lax.fori_loop(0, _ROWS_PER_W // _GCH, step, 0)


_gather = pl.kernel(
    _gather_body,
    out_type=jax.ShapeDtypeStruct((TOPK * Q, C), jnp.float32),
    mesh=plsc.VectorSubcoreMesh(core_axis_name="c", subcore_axis_name="s"),
    scratch_types=[
        pltpu.VMEM((_GCH,), jnp.int32),
        pltpu.VMEM((_GCH, C), jnp.float32),
        pltpu.SemaphoreType.DMA,
    ],
)


BQ3 = 256
NB3 = Q // BQ3
N3 = BQ3 * C


def _mlp_body(g_ref, w1t_ref, b1_ref, w2t_ref, b2_ref, o_ref):
    h = jnp.dot(w1t_ref[...], g_ref[...])
    h = jnp.maximum(h + b1_ref[...], 0.0)
    o = jnp.dot(w2t_ref[...], h) + b2_ref[...]
    o_ref[...] = o.reshape(1, 1, N3)


def _mlp_call(gflat, w1t, b1c, w2t, b2c):
    return pl.pallas_call(
        _mlp_body,
        grid=(NB3,),
        in_specs=[
            pl.BlockSpec((TOPK, N3), lambda i: (0, i)),
            pl.BlockSpec((TOPK, TOPK), lambda i: (0, 0)),
            pl.BlockSpec((TOPK, 1), lambda i: (0, 0)),
            pl.BlockSpec((1, TOPK), lambda i: (0, 0)),
            pl.BlockSpec((1, 1), lambda i: (0, 0)),
        ],
        out_specs=pl.BlockSpec((1, 1, N3), lambda i: (i, 0, 0)),
        out_shape=jax.ShapeDtypeStruct((NB3, 1, N3), jnp.float32),
    )(gflat, w1t, b1c, w2t, b2c)


def kernel(queries, keys, point_feats, bev_feats, W1, b1, W2, b2, R, t):
    proj = keys @ R.T + t
    qq = jnp.sum(queries * queries, axis=1, keepdims=True)
    pp = jnp.sum(proj * proj, axis=1)[None, :]
    ptc = proj.T.reshape(3, NCH, CHK).transpose(1, 0, 2)
    ppc = pp.reshape(1, NCH, CHK).transpose(1, 0, 2)

    idx = _topk_call(queries, ptc, qq, ppc)
    idx_flat = idx.T.reshape(TOPK * Q)

    g = _gather(point_feats, idx_flat)
    gflat = g.reshape(TOPK, Q * C)

    hb = _mlp_call(gflat, W1.T, b1.reshape(TOPK, 1),
                   W2.T, b2.reshape(1, 1))
    h = hb.reshape(Q, C)
    return jnp.concatenate([bev_feats, h], axis=-1)
